# baseline (speedup 1.0000x reference)
import jax
import jax.numpy as jnp
from jax.experimental import pallas as pl
from jax.experimental.pallas import tpu as pltpu

N = 4096
D = 128
UNITS = 128
T = 512
NT = N // T


def _gat_kernel(x_ref, adj_ref, wpre_ref, watt_ref, out_ref,
                h_ref, g_ref, acc_ref, w_ref, hs_ref):
    j = pl.program_id(0)

    @pl.when(j == 0)
    def _init():
        h = jnp.dot(x_ref[...], wpre_ref[...],
                    preferred_element_type=jnp.float32)
        h_ref[...] = h
        wa = watt_ref[...]
        w2 = jnp.concatenate([wa[:UNITS, :], wa[UNITS:, :]], axis=1)
        g_ref[...] = jnp.dot(h, w2, preferred_element_type=jnp.float32)
        acc_ref[...] = jnp.zeros_like(acc_ref)
        w_ref[1] = jnp.zeros_like(w_ref[1])
        hs_ref[1] = jnp.zeros_like(hs_ref[1])

    if True:
        es_col = g_ref[:, 0:1]
        er_row = g_ref[pl.ds(j * T, T), 1:2].T
        ms = jnp.max(es_col)
        tm = ms + er_row
        m = jnp.maximum(tm, 0.2 * tm)
        a1 = jnp.exp(es_col - ms)
        b1 = jnp.exp(er_row + (ms - m))
        a2 = jnp.exp(0.2 * (es_col - ms))
        b2 = jnp.exp(0.2 * er_row + (0.2 * ms - m))

        w = adj_ref[...] * jnp.maximum(a1 * b1, a2 * b2)

        eye = (jax.lax.broadcasted_iota(jnp.int32, (T, T), 0)
               == jax.lax.broadcasted_iota(jnp.int32, (T, T), 1))
        adj_diag = jnp.sum(jnp.where(eye, adj_ref[pl.ds(j * T, T), :], 0.0),
                           axis=0, keepdims=True)
        es_strip = g_ref[pl.ds(j * T, T), 0:1].T
        val_diag = jnp.maximum(jnp.exp(es_strip - ms) * b1,
                               jnp.exp(0.2 * (es_strip - ms)) * b2)
        corr = (1.0 - adj_diag) * val_diag

        z = jnp.sum(w, axis=0, keepdims=True) + corr
        h_tile = h_ref[pl.ds(j * T, T), :] * (1.0 / z).T
        w_ref[j % 2] = w
        hs_ref[j % 2] = h_tile
        acc_ref[pl.ds(j * T, T), :] += corr.T * h_tile

    p = (j - 1) % 2
    acc_ref[...] += jnp.dot(w_ref[p], hs_ref[p],
                            preferred_element_type=jnp.float32)

    @pl.when(j == NT - 1)
    def _fin():
        q = (NT - 1) % 2
        a = acc_ref[...] + jnp.dot(w_ref[q], hs_ref[q],
                                   preferred_element_type=jnp.float32)
        out_ref[...] = jnp.where(a > 0, a, jnp.exp(a) - 1.0)


@jax.jit
def kernel(x, adj, W_pre, W_att):
    out = pl.pallas_call(
        _gat_kernel,
        grid=(NT,),
        in_specs=[
            pl.BlockSpec((N, D), lambda j: (0, 0)),
            pl.BlockSpec((N, T), lambda j: (0, j)),
            pl.BlockSpec((D, UNITS), lambda j: (0, 0)),
            pl.BlockSpec((2 * UNITS, 1), lambda j: (0, 0)),
        ],
        out_specs=pl.BlockSpec((N, UNITS), lambda j: (0, 0)),
        out_shape=jax.ShapeDtypeStruct((N, UNITS), jnp.float32),
        scratch_shapes=[
            pltpu.VMEM((N, UNITS), jnp.float32),
            pltpu.VMEM((N, 2), jnp.float32),
            pltpu.VMEM((N, UNITS), jnp.float32),
            pltpu.VMEM((2, N, T), jnp.float32),
            pltpu.VMEM((2, T, UNITS), jnp.float32),
        ],
    )(x[0], adj[0], W_pre, W_att)
    return out[None]


# speedup vs baseline: 1.1156x; 1.1156x over previous
import jax
import jax.numpy as jnp
from jax.experimental import pallas as pl
from jax.experimental.pallas import tpu as pltpu

N = 4096
D = 128
UNITS = 128
T = 512
NT = N // T


def _gat_kernel(x_ref, adj_ref, wpre_ref, watt_ref, out_ref,
                h_ref, g_ref, acc_ref, w_ref, hs_ref):
    j = pl.program_id(0)

    @pl.when(j == 0)
    def _init():
        h = jnp.dot(x_ref[...], wpre_ref[...],
                    preferred_element_type=jnp.float32)
        h_ref[...] = h
        wa = watt_ref[...]
        w2 = jnp.concatenate([wa[:UNITS, :], wa[UNITS:, :]], axis=1)
        g_ref[...] = jnp.dot(h, w2, preferred_element_type=jnp.float32)
        acc_ref[...] = jnp.zeros_like(acc_ref)
        w_ref[1] = jnp.zeros_like(w_ref[1])
        hs_ref[1] = jnp.zeros_like(hs_ref[1])

    p = (j - 1) % 2
    acc_ref[...] += jnp.dot(w_ref[p], hs_ref[p],
                            preferred_element_type=jnp.float32)

    if True:
        es_col = g_ref[:, 0:1]
        er_row = g_ref[pl.ds(j * T, T), 1:2].T
        ms = jnp.max(es_col)
        tm = ms + er_row
        m = jnp.maximum(tm, 0.2 * tm)
        a1 = jnp.exp(es_col - ms)
        b1 = jnp.exp(er_row + (ms - m))
        a2 = jnp.exp(0.2 * (es_col - ms))
        b2 = jnp.exp(0.2 * er_row + (0.2 * ms - m))

        w = adj_ref[...] * jnp.maximum(a1 * b1, a2 * b2)

        eye = (jax.lax.broadcasted_iota(jnp.int32, (T, T), 0)
               == jax.lax.broadcasted_iota(jnp.int32, (T, T), 1))
        adj_diag = jnp.sum(jnp.where(eye, adj_ref[pl.ds(j * T, T), :], 0.0),
                           axis=0, keepdims=True)
        es_strip = g_ref[pl.ds(j * T, T), 0:1].T
        val_diag = jnp.maximum(jnp.exp(es_strip - ms) * b1,
                               jnp.exp(0.2 * (es_strip - ms)) * b2)
        corr = (1.0 - adj_diag) * val_diag

        z = jnp.sum(w, axis=0, keepdims=True) + corr
        h_tile = h_ref[pl.ds(j * T, T), :] * (1.0 / z).T
        w_ref[j % 2] = w.astype(jnp.bfloat16)
        hs_ref[j % 2] = h_tile.astype(jnp.bfloat16)
        acc_ref[pl.ds(j * T, T), :] += corr.T * h_tile

    @pl.when(j == NT - 1)
    def _fin():
        q = (NT - 1) % 2
        a = acc_ref[...] + jnp.dot(w_ref[q], hs_ref[q],
                                   preferred_element_type=jnp.float32)
        out_ref[...] = jnp.where(a > 0, a, jnp.exp(a) - 1.0)


@jax.jit
def kernel(x, adj, W_pre, W_att):
    out = pl.pallas_call(
        _gat_kernel,
        grid=(NT,),
        in_specs=[
            pl.BlockSpec((N, D), lambda j: (0, 0)),
            pl.BlockSpec((N, T), lambda j: (0, j)),
            pl.BlockSpec((D, UNITS), lambda j: (0, 0)),
            pl.BlockSpec((2 * UNITS, 1), lambda j: (0, 0)),
        ],
        out_specs=pl.BlockSpec((N, UNITS), lambda j: (0, 0)),
        out_shape=jax.ShapeDtypeStruct((N, UNITS), jnp.float32),
        scratch_shapes=[
            pltpu.VMEM((N, UNITS), jnp.float32),
            pltpu.VMEM((N, 2), jnp.float32),
            pltpu.VMEM((N, UNITS), jnp.float32),
            pltpu.VMEM((2, N, T), jnp.bfloat16),
            pltpu.VMEM((2, T, UNITS), jnp.bfloat16),
        ],
    )(x[0], adj[0], W_pre, W_att)
    return out[None]
